# baseline (device time: 210586 ns/iter reference)
import jax
import jax.numpy as jnp
from jax import lax
from jax.experimental import pallas as pl
from jax.experimental.pallas import tpu as pltpu

N_DEV = 32
M = 1024
CHUNK = M // N_DEV


def kernel(x, w_mat):
    m, k_per = x.shape
    _, n = w_mat.shape

    def body(x_ref, w_ref, out_ref, rs_recv, rs_recv_sems, ag_recv_sems, send_sem):
        i = lax.axis_index("i")
        left = lax.rem(i - 1 + N_DEV, N_DEV)
        right = lax.rem(i + 1, N_DEV)

        barrier_sem = pltpu.get_barrier_semaphore()
        for nbr in (left, right):
            pl.semaphore_signal(
                barrier_sem, inc=1,
                device_id=(nbr,), device_id_type=pl.DeviceIdType.MESH,
            )
        pl.semaphore_wait(barrier_sem, 2)

        out_ref[...] = jnp.dot(
            x_ref[...], w_ref[...], preferred_element_type=jnp.float32
        )

        for h in range(N_DEV - 1):
            s = lax.rem(i - h + N_DEV, N_DEV)
            rdma = pltpu.make_async_remote_copy(
                src_ref=out_ref.at[pl.ds(s * CHUNK, CHUNK), :],
                dst_ref=rs_recv.at[h],
                send_sem=send_sem,
                recv_sem=rs_recv_sems.at[h],
                device_id=(right,),
                device_id_type=pl.DeviceIdType.MESH,
            )
            rdma.start()
            rdma.wait()
            r = lax.rem(i - h - 1 + N_DEV, N_DEV)
            out_ref[pl.ds(r * CHUNK, CHUNK), :] = (
                out_ref[pl.ds(r * CHUNK, CHUNK), :] + rs_recv[h]
            )

        own = lax.rem(i + 1, N_DEV)
        yc = out_ref[pl.ds(own * CHUNK, CHUNK), :]
        out_ref[pl.ds(own * CHUNK, CHUNK), :] = yc * jax.nn.sigmoid(yc)

        for h in range(N_DEV - 1):
            c = lax.rem(i + 1 - h + N_DEV, N_DEV)
            rdma = pltpu.make_async_remote_copy(
                src_ref=out_ref.at[pl.ds(c * CHUNK, CHUNK), :],
                dst_ref=out_ref.at[pl.ds(c * CHUNK, CHUNK), :],
                send_sem=send_sem,
                recv_sem=ag_recv_sems.at[h],
                device_id=(right,),
                device_id_type=pl.DeviceIdType.MESH,
            )
            rdma.start()
            rdma.wait()

    return pl.pallas_call(
        body,
        out_shape=jax.ShapeDtypeStruct((m, n), jnp.float32),
        in_specs=[
            pl.BlockSpec(memory_space=pltpu.VMEM),
            pl.BlockSpec(memory_space=pltpu.VMEM),
        ],
        out_specs=pl.BlockSpec(memory_space=pltpu.VMEM),
        scratch_shapes=[
            pltpu.VMEM((N_DEV - 1, CHUNK, n), jnp.float32),
            pltpu.SemaphoreType.DMA((N_DEV - 1,)),
            pltpu.SemaphoreType.DMA((N_DEV - 1,)),
            pltpu.SemaphoreType.DMA,
        ],
        compiler_params=pltpu.CompilerParams(collective_id=0),
    )(x, w_mat)


# device time: 117898 ns/iter; 1.7862x vs baseline; 1.7862x over previous
import jax
import jax.numpy as jnp
from jax import lax
from jax.experimental import pallas as pl
from jax.experimental.pallas import tpu as pltpu

N_DEV = 32
M = 1024
CHUNK = M // N_DEV


def kernel(x, w_mat):
    m, k_per = x.shape
    _, n = w_mat.shape

    def body(x_ref, w_ref, out_ref, rs_recv,
             rs_ssem, rs_rsem, ag_ssem, ag_rsem):
        i = lax.axis_index("i")

        barrier_sem = pltpu.get_barrier_semaphore()
        for o in range(1, N_DEV):
            peer = lax.rem(i + o, N_DEV)
            pl.semaphore_signal(
                barrier_sem, inc=1,
                device_id=(peer,), device_id_type=pl.DeviceIdType.MESH,
            )
        pl.semaphore_wait(barrier_sem, N_DEV - 1)

        out_ref[...] = jnp.dot(
            x_ref[...], w_ref[...], preferred_element_type=jnp.float32
        )

        rs = []
        for o in range(1, N_DEV):
            d = lax.rem(i + o, N_DEV)
            rdma = pltpu.make_async_remote_copy(
                src_ref=out_ref.at[pl.ds(d * CHUNK, CHUNK), :],
                dst_ref=rs_recv.at[o - 1],
                send_sem=rs_ssem.at[o - 1],
                recv_sem=rs_rsem.at[o - 1],
                device_id=(d,),
                device_id_type=pl.DeviceIdType.MESH,
            )
            rdma.start()
            rs.append(rdma)
        for rdma in rs:
            rdma.wait()

        acc = out_ref[pl.ds(i * CHUNK, CHUNK), :] + jnp.sum(
            rs_recv[...], axis=0
        )
        out_ref[pl.ds(i * CHUNK, CHUNK), :] = acc * jax.nn.sigmoid(acc)

        ag = []
        for o in range(1, N_DEV):
            d = lax.rem(i + o, N_DEV)
            rdma = pltpu.make_async_remote_copy(
                src_ref=out_ref.at[pl.ds(i * CHUNK, CHUNK), :],
                dst_ref=out_ref.at[pl.ds(i * CHUNK, CHUNK), :],
                send_sem=ag_ssem.at[o - 1],
                recv_sem=ag_rsem.at[o - 1],
                device_id=(d,),
                device_id_type=pl.DeviceIdType.MESH,
            )
            rdma.start()
            ag.append(rdma)
        for rdma in ag:
            rdma.wait()

    return pl.pallas_call(
        body,
        out_shape=jax.ShapeDtypeStruct((m, n), jnp.float32),
        in_specs=[
            pl.BlockSpec(memory_space=pltpu.VMEM),
            pl.BlockSpec(memory_space=pltpu.VMEM),
        ],
        out_specs=pl.BlockSpec(memory_space=pltpu.VMEM),
        scratch_shapes=[
            pltpu.VMEM((N_DEV - 1, CHUNK, n), jnp.float32),
            pltpu.SemaphoreType.DMA((N_DEV - 1,)),
            pltpu.SemaphoreType.DMA((N_DEV - 1,)),
            pltpu.SemaphoreType.DMA((N_DEV - 1,)),
            pltpu.SemaphoreType.DMA((N_DEV - 1,)),
        ],
        compiler_params=pltpu.CompilerParams(collective_id=0),
    )(x, w_mat)


# device time: 107325 ns/iter; 1.9621x vs baseline; 1.0985x over previous
import jax
import jax.numpy as jnp
from jax import lax
from jax.experimental import pallas as pl
from jax.experimental.pallas import tpu as pltpu

N_DEV = 32
M = 1024
CHUNK = M // N_DEV
N_ROUNDS = 2
SUB = CHUNK // N_ROUNDS


def kernel(x, w_mat):
    m, k_per = x.shape
    _, n = w_mat.shape

    def body(x_ref, w_ref, out_ref, rs_recv,
             rs_ssem, rs_rsem, ag_ssem, ag_rsem):
        i = lax.axis_index("i")

        barrier_sem = pltpu.get_barrier_semaphore()
        for o in range(1, N_DEV):
            peer = lax.rem(i + o, N_DEV)
            pl.semaphore_signal(
                barrier_sem, inc=1,
                device_id=(peer,), device_id_type=pl.DeviceIdType.MESH,
            )
        pl.semaphore_wait(barrier_sem, N_DEV - 1)

        out_ref[...] = jnp.dot(
            x_ref[...], w_ref[...], preferred_element_type=jnp.float32
        )

        rs = [[None] * (N_DEV - 1) for _ in range(N_ROUNDS)]
        for r in range(N_ROUNDS):
            for o in range(1, N_DEV):
                d = lax.rem(i + o, N_DEV)
                rdma = pltpu.make_async_remote_copy(
                    src_ref=out_ref.at[pl.ds(d * CHUNK + r * SUB, SUB), :],
                    dst_ref=rs_recv.at[o - 1, pl.ds(r * SUB, SUB), :],
                    send_sem=rs_ssem.at[r, o - 1],
                    recv_sem=rs_rsem.at[r, o - 1],
                    device_id=(d,),
                    device_id_type=pl.DeviceIdType.MESH,
                )
                rdma.start()
                rs[r][o - 1] = rdma

        ag = [[None] * (N_DEV - 1) for _ in range(N_ROUNDS)]
        for r in range(N_ROUNDS):
            for rdma in rs[r]:
                rdma.wait_recv()
            rows = pl.ds(i * CHUNK + r * SUB, SUB)
            acc = out_ref[rows, :] + jnp.sum(
                rs_recv[:, pl.ds(r * SUB, SUB), :], axis=0
            )
            out_ref[rows, :] = acc * jax.nn.sigmoid(acc)
            for o in range(1, N_DEV):
                d = lax.rem(i + o, N_DEV)
                rdma = pltpu.make_async_remote_copy(
                    src_ref=out_ref.at[rows, :],
                    dst_ref=out_ref.at[rows, :],
                    send_sem=ag_ssem.at[r, o - 1],
                    recv_sem=ag_rsem.at[r, o - 1],
                    device_id=(d,),
                    device_id_type=pl.DeviceIdType.MESH,
                )
                rdma.start()
                ag[r][o - 1] = rdma

        for r in range(N_ROUNDS):
            for rdma in ag[r]:
                rdma.wait()
            for rdma in rs[r]:
                rdma.wait_send()

    return pl.pallas_call(
        body,
        out_shape=jax.ShapeDtypeStruct((m, n), jnp.float32),
        in_specs=[
            pl.BlockSpec(memory_space=pltpu.VMEM),
            pl.BlockSpec(memory_space=pltpu.VMEM),
        ],
        out_specs=pl.BlockSpec(memory_space=pltpu.VMEM),
        scratch_shapes=[
            pltpu.VMEM((N_DEV - 1, CHUNK, n), jnp.float32),
            pltpu.SemaphoreType.DMA((N_ROUNDS, N_DEV - 1)),
            pltpu.SemaphoreType.DMA((N_ROUNDS, N_DEV - 1)),
            pltpu.SemaphoreType.DMA((N_ROUNDS, N_DEV - 1)),
            pltpu.SemaphoreType.DMA((N_ROUNDS, N_DEV - 1)),
        ],
        compiler_params=pltpu.CompilerParams(collective_id=0),
    )(x, w_mat)


# device time: 65509 ns/iter; 3.2146x vs baseline; 1.6383x over previous
import jax
import jax.numpy as jnp
from jax import lax
from jax.experimental import pallas as pl
from jax.experimental.pallas import tpu as pltpu

N_DEV = 32
M = 1024
CHUNK = M // N_DEV
N_ROUNDS = 1
SUB = CHUNK // N_ROUNDS


def kernel(x, w_mat):
    m, k_per = x.shape
    _, n = w_mat.shape

    def body(x_ref, w_ref, out_ref, part_bf, ag_bf, rs_recv,
             rs_ssem, rs_rsem, ag_ssem, ag_rsem):
        i = lax.axis_index("i")

        barrier_sem = pltpu.get_barrier_semaphore()
        for o in range(1, N_DEV):
            peer = lax.rem(i + o, N_DEV)
            pl.semaphore_signal(
                barrier_sem, inc=1,
                device_id=(peer,), device_id_type=pl.DeviceIdType.MESH,
            )
        pl.semaphore_wait(barrier_sem, N_DEV - 1)

        out_ref[...] = jnp.dot(
            x_ref[...], w_ref[...], preferred_element_type=jnp.float32
        )
        part_bf[...] = out_ref[...].astype(jnp.bfloat16)

        rs = [[None] * (N_DEV - 1) for _ in range(N_ROUNDS)]
        for r in range(N_ROUNDS):
            for o in range(1, N_DEV):
                d = lax.rem(i + o, N_DEV)
                rdma = pltpu.make_async_remote_copy(
                    src_ref=part_bf.at[pl.ds(d * CHUNK + r * SUB, SUB), :],
                    dst_ref=rs_recv.at[o - 1, pl.ds(r * SUB, SUB), :],
                    send_sem=rs_ssem.at[r, o - 1],
                    recv_sem=rs_rsem.at[r, o - 1],
                    device_id=(d,),
                    device_id_type=pl.DeviceIdType.MESH,
                )
                rdma.start()
                rs[r][o - 1] = rdma

        ag = [[None] * (N_DEV - 1) for _ in range(N_ROUNDS)]
        for r in range(N_ROUNDS):
            for rdma in rs[r]:
                rdma.wait_recv()
            rows = pl.ds(i * CHUNK + r * SUB, SUB)
            acc = out_ref[rows, :] + jnp.sum(
                rs_recv[:, pl.ds(r * SUB, SUB), :].astype(jnp.float32),
                axis=0,
            )
            ag_bf[rows, :] = (acc * jax.nn.sigmoid(acc)).astype(jnp.bfloat16)
            for o in range(1, N_DEV):
                d = lax.rem(i + o, N_DEV)
                rdma = pltpu.make_async_remote_copy(
                    src_ref=ag_bf.at[rows, :],
                    dst_ref=ag_bf.at[rows, :],
                    send_sem=ag_ssem.at[r, o - 1],
                    recv_sem=ag_rsem.at[r, o - 1],
                    device_id=(d,),
                    device_id_type=pl.DeviceIdType.MESH,
                )
                rdma.start()
                ag[r][o - 1] = rdma

        for r in range(N_ROUNDS):
            for rdma in ag[r]:
                rdma.wait()
            for rdma in rs[r]:
                rdma.wait_send()

        out_ref[...] = ag_bf[...].astype(jnp.float32)

    return pl.pallas_call(
        body,
        out_shape=jax.ShapeDtypeStruct((m, n), jnp.float32),
        in_specs=[
            pl.BlockSpec(memory_space=pltpu.VMEM),
            pl.BlockSpec(memory_space=pltpu.VMEM),
        ],
        out_specs=pl.BlockSpec(memory_space=pltpu.VMEM),
        scratch_shapes=[
            pltpu.VMEM((m, n), jnp.bfloat16),
            pltpu.VMEM((m, n), jnp.bfloat16),
            pltpu.VMEM((N_DEV - 1, CHUNK, n), jnp.bfloat16),
            pltpu.SemaphoreType.DMA((N_ROUNDS, N_DEV - 1)),
            pltpu.SemaphoreType.DMA((N_ROUNDS, N_DEV - 1)),
            pltpu.SemaphoreType.DMA((N_ROUNDS, N_DEV - 1)),
            pltpu.SemaphoreType.DMA((N_ROUNDS, N_DEV - 1)),
        ],
        compiler_params=pltpu.CompilerParams(collective_id=0),
    )(x, w_mat)
